# baseline (speedup 1.0000x reference)
import functools

import jax
import jax.numpy as jnp
from jax import lax
from jax.experimental import pallas as pl
from jax.experimental.pallas import tpu as pltpu
from jax.experimental.pallas import tpu_sc as plsc

B = 9216
D = 256
NKEYS = 1024
BLK = 512

STRIPES = (2048, 2560, 2560, 2048)
OFFSETS = (0, 2048, 4608, 7168)

NC, NS = 2, 16
NW = NC * NS


def _argmin_first_body(f_ref, k_ref, out_ref, knorm_out):
    kw = k_ref[...]
    @pl.when(pl.program_id(0) == 0)
    def _():
        knorm_out[...] = jnp.sum(kw * kw, axis=1)[None, :]

    f = f_ref[...]
    mm = lax.dot_general(f, kw, (((1,), (1,)), ((), ())),
                         preferred_element_type=jnp.float32)
    fnorm = jnp.sum(f * f, axis=1, keepdims=True)
    d = (fnorm + knorm_out[...]) - 2.0 * mm
    out_ref[...] = jnp.argmin(d, axis=1).astype(jnp.int32)


def _argmin_rest_body(f_ref, k_ref, knorm_ref, out_ref):
    f = f_ref[...]
    kw = k_ref[...]
    mm = lax.dot_general(f, kw, (((1,), (1,)), ((), ())),
                         preferred_element_type=jnp.float32)
    fnorm = jnp.sum(f * f, axis=1, keepdims=True)
    d = (fnorm + knorm_ref[...]) - 2.0 * mm
    out_ref[...] = jnp.argmin(d, axis=1).astype(jnp.int32)


def _argmin_tc_first(flat, key_weights):
    nblk = STRIPES[0] // BLK
    idx, knorm = pl.pallas_call(
        _argmin_first_body,
        grid=(nblk,),
        in_specs=[
            pl.BlockSpec((BLK, D), lambda i: (i, 0)),
            pl.BlockSpec((NKEYS, D), lambda i: (0, 0)),
        ],
        out_specs=[
            pl.BlockSpec((BLK,), lambda i: (i,)),
            pl.BlockSpec((1, NKEYS), lambda i: (0, 0)),
        ],
        out_shape=[
            jax.ShapeDtypeStruct((STRIPES[0],), jnp.int32),
            jax.ShapeDtypeStruct((1, NKEYS), jnp.float32),
        ],
    )(flat, key_weights)
    return idx, knorm


def _argmin_tc_rest(flat, key_weights, knorm, stripe):
    nblk = STRIPES[stripe] // BLK
    blk_off = OFFSETS[stripe] // BLK
    idx = pl.pallas_call(
        _argmin_rest_body,
        grid=(nblk,),
        in_specs=[
            pl.BlockSpec((BLK, D), lambda i: (i + blk_off, 0)),
            pl.BlockSpec((NKEYS, D), lambda i: (0, 0)),
            pl.BlockSpec((1, NKEYS), lambda i: (0, 0)),
        ],
        out_specs=pl.BlockSpec((BLK,), lambda i: (i,)),
        out_shape=jax.ShapeDtypeStruct((STRIPES[stripe],), jnp.int32),
    )(flat, key_weights, knorm)
    return idx


@functools.cache
def _make_gather_add_sc(stripe):
    b_s = STRIPES[stripe]
    chunk = b_s // NW

    @functools.partial(
        pl.kernel,
        mesh=plsc.VectorSubcoreMesh(core_axis_name="c", subcore_axis_name="s"),
        scratch_types=[
            pltpu.VMEM((chunk,), jnp.int32),
            pltpu.VMEM((chunk, D), jnp.float32),
            pltpu.VMEM((chunk, D), jnp.float32),
            pltpu.SemaphoreType.DMA,
            pltpu.SemaphoreType.DMA,
        ],
    )
    def _gather_add_sc(flat_hbm, idx_hbm, val_hbm, out_hbm, idx_v, rows_v,
                       flat_v, gsem, fsem):
        wid = lax.axis_index("s") * NC + lax.axis_index("c")
        base = wid * chunk
        fcopy = pltpu.async_copy(
            flat_hbm.at[pl.ds(OFFSETS[stripe] + base, chunk)], flat_v, fsem)
        pltpu.sync_copy(idx_hbm.at[pl.ds(base, chunk)], idx_v)
        gather = pltpu.async_copy(val_hbm.at[idx_v], rows_v, gsem)
        fcopy.wait()
        gather.wait()

        @plsc.parallel_loop(0, chunk, 1, unroll=4)
        def _add(r):
            for j in range(D // 16):
                sl = pl.ds(j * 16, 16)
                rows_v[r, sl] = rows_v[r, sl] + flat_v[r, sl]

        pltpu.sync_copy(
            rows_v, out_hbm.at[pl.ds(OFFSETS[stripe] + base, chunk)])

    return _gather_add_sc


def kernel(inputs, key_weights, value_weights):
    size = inputs.shape
    flat = inputs.reshape(-1, D)
    out_ref = jax.new_ref(jnp.zeros((B, D), jnp.float32))
    idx0, knorm = _argmin_tc_first(flat, key_weights)
    _make_gather_add_sc(0)(flat, idx0, value_weights, out_ref)
    for r in range(1, len(STRIPES)):
        idx = _argmin_tc_rest(flat, key_weights, knorm, r)
        _make_gather_add_sc(r)(flat, idx, value_weights, out_ref)
    return out_ref[...].reshape(size)


# speedup vs baseline: 1.0549x; 1.0549x over previous
import functools

import jax
import jax.numpy as jnp
from jax import lax
from jax.experimental import pallas as pl
from jax.experimental.pallas import tpu as pltpu
from jax.experimental.pallas import tpu_sc as plsc

B = 9216
D = 256
NKEYS = 1024
BLK = 512

STRIPES = (2048, 4096, 3072)
OFFSETS = (0, 2048, 6144)

NC, NS = 2, 16
NW = NC * NS


def _argmin_first_body(f_ref, k_ref, out_ref, knorm_out):
    kw = k_ref[...]
    @pl.when(pl.program_id(0) == 0)
    def _():
        knorm_out[...] = jnp.sum(kw * kw, axis=1)[None, :]

    f = f_ref[...]
    mm = lax.dot_general(f, kw, (((1,), (1,)), ((), ())),
                         preferred_element_type=jnp.float32)
    fnorm = jnp.sum(f * f, axis=1, keepdims=True)
    d = (fnorm + knorm_out[...]) - 2.0 * mm
    out_ref[...] = jnp.argmin(d, axis=1).astype(jnp.int32)


def _argmin_rest_body(f_ref, k_ref, knorm_ref, out_ref):
    f = f_ref[...]
    kw = k_ref[...]
    mm = lax.dot_general(f, kw, (((1,), (1,)), ((), ())),
                         preferred_element_type=jnp.float32)
    fnorm = jnp.sum(f * f, axis=1, keepdims=True)
    d = (fnorm + knorm_ref[...]) - 2.0 * mm
    out_ref[...] = jnp.argmin(d, axis=1).astype(jnp.int32)


def _argmin_tc_first(flat, key_weights):
    nblk = STRIPES[0] // BLK
    idx, knorm = pl.pallas_call(
        _argmin_first_body,
        grid=(nblk,),
        in_specs=[
            pl.BlockSpec((BLK, D), lambda i: (i, 0)),
            pl.BlockSpec((NKEYS, D), lambda i: (0, 0)),
        ],
        out_specs=[
            pl.BlockSpec((BLK,), lambda i: (i,)),
            pl.BlockSpec((1, NKEYS), lambda i: (0, 0)),
        ],
        out_shape=[
            jax.ShapeDtypeStruct((STRIPES[0],), jnp.int32),
            jax.ShapeDtypeStruct((1, NKEYS), jnp.float32),
        ],
    )(flat, key_weights)
    return idx, knorm


def _argmin_tc_rest(flat, key_weights, knorm, stripe):
    nblk = STRIPES[stripe] // BLK
    blk_off = OFFSETS[stripe] // BLK
    idx = pl.pallas_call(
        _argmin_rest_body,
        grid=(nblk,),
        in_specs=[
            pl.BlockSpec((BLK, D), lambda i: (i + blk_off, 0)),
            pl.BlockSpec((NKEYS, D), lambda i: (0, 0)),
            pl.BlockSpec((1, NKEYS), lambda i: (0, 0)),
        ],
        out_specs=pl.BlockSpec((BLK,), lambda i: (i,)),
        out_shape=jax.ShapeDtypeStruct((STRIPES[stripe],), jnp.int32),
    )(flat, key_weights, knorm)
    return idx


@functools.cache
def _make_gather_add_sc(stripe):
    b_s = STRIPES[stripe]
    chunk = b_s // NW

    @functools.partial(
        pl.kernel,
        mesh=plsc.VectorSubcoreMesh(core_axis_name="c", subcore_axis_name="s"),
        scratch_types=[
            pltpu.VMEM((chunk,), jnp.int32),
            pltpu.VMEM((chunk, D), jnp.float32),
            pltpu.VMEM((chunk, D), jnp.float32),
            pltpu.SemaphoreType.DMA,
            pltpu.SemaphoreType.DMA,
        ],
    )
    def _gather_add_sc(flat_hbm, idx_hbm, val_hbm, out_hbm, idx_v, rows_v,
                       flat_v, gsem, fsem):
        wid = lax.axis_index("s") * NC + lax.axis_index("c")
        base = wid * chunk
        fcopy = pltpu.async_copy(
            flat_hbm.at[pl.ds(OFFSETS[stripe] + base, chunk)], flat_v, fsem)
        pltpu.sync_copy(idx_hbm.at[pl.ds(base, chunk)], idx_v)
        gather = pltpu.async_copy(val_hbm.at[idx_v], rows_v, gsem)
        fcopy.wait()
        gather.wait()

        @plsc.parallel_loop(0, chunk, 1, unroll=4)
        def _add(r):
            for j in range(D // 16):
                sl = pl.ds(j * 16, 16)
                rows_v[r, sl] = rows_v[r, sl] + flat_v[r, sl]

        pltpu.sync_copy(
            rows_v, out_hbm.at[pl.ds(OFFSETS[stripe] + base, chunk)])

    return _gather_add_sc


def kernel(inputs, key_weights, value_weights):
    size = inputs.shape
    flat = inputs.reshape(-1, D)
    out_ref = jax.new_ref(jnp.zeros((B, D), jnp.float32))
    idx0, knorm = _argmin_tc_first(flat, key_weights)
    _make_gather_add_sc(0)(flat, idx0, value_weights, out_ref)
    for r in range(1, len(STRIPES)):
        idx = _argmin_tc_rest(flat, key_weights, knorm, r)
        _make_gather_add_sc(r)(flat, idx, value_weights, out_ref)
    return out_ref[...].reshape(size)


# speedup vs baseline: 1.1062x; 1.0486x over previous
import functools

import jax
import jax.numpy as jnp
from jax import lax
from jax.experimental import pallas as pl
from jax.experimental.pallas import tpu as pltpu
from jax.experimental.pallas import tpu_sc as plsc

B = 9216
D = 256
NKEYS = 1024
BLK = 512

STRIPES = (2048, 4096, 3072)
OFFSETS = (0, 2048, 6144)

NC, NS = 2, 16
NW = NC * NS


def _argmin_first_body(f_ref, k_ref, out_ref, knorm_out):
    kw = k_ref[...]
    @pl.when(pl.program_id(0) == 0)
    def _():
        knorm_out[...] = jnp.sum(kw * kw, axis=1)[None, :]

    f = f_ref[...]
    mm = lax.dot_general(f, kw, (((1,), (1,)), ((), ())),
                         preferred_element_type=jnp.float32)
    fnorm = jnp.sum(f * f, axis=1, keepdims=True)
    d = (fnorm + knorm_out[...]) - 2.0 * mm
    out_ref[...] = jnp.argmin(d, axis=1).astype(jnp.int32)


def _argmin_rest_body(f_ref, k_ref, knorm_ref, out_ref):
    f = f_ref[...]
    kw = k_ref[...]
    mm = lax.dot_general(f, kw, (((1,), (1,)), ((), ())),
                         preferred_element_type=jnp.float32)
    fnorm = jnp.sum(f * f, axis=1, keepdims=True)
    d = (fnorm + knorm_ref[...]) - 2.0 * mm
    out_ref[...] = jnp.argmin(d, axis=1).astype(jnp.int32)


def _argmin_tc_first(flat, key_weights):
    nblk = STRIPES[0] // BLK
    idx, knorm = pl.pallas_call(
        _argmin_first_body,
        grid=(nblk,),
        in_specs=[
            pl.BlockSpec((BLK, D), lambda i: (i, 0)),
            pl.BlockSpec((NKEYS, D), lambda i: (0, 0)),
        ],
        out_specs=[
            pl.BlockSpec((BLK,), lambda i: (i,)),
            pl.BlockSpec((1, NKEYS), lambda i: (0, 0)),
        ],
        out_shape=[
            jax.ShapeDtypeStruct((STRIPES[0],), jnp.int32),
            jax.ShapeDtypeStruct((1, NKEYS), jnp.float32),
        ],
    )(flat, key_weights)
    return idx, knorm


def _argmin_tc_rest(flat, key_weights, knorm, stripe):
    nblk = STRIPES[stripe] // BLK
    blk_off = OFFSETS[stripe] // BLK
    idx = pl.pallas_call(
        _argmin_rest_body,
        grid=(nblk,),
        in_specs=[
            pl.BlockSpec((BLK, D), lambda i: (i + blk_off, 0)),
            pl.BlockSpec((NKEYS, D), lambda i: (0, 0)),
            pl.BlockSpec((1, NKEYS), lambda i: (0, 0)),
        ],
        out_specs=pl.BlockSpec((BLK,), lambda i: (i,)),
        out_shape=jax.ShapeDtypeStruct((STRIPES[stripe],), jnp.int32),
    )(flat, key_weights, knorm)
    return idx


@functools.cache
def _make_gather_add_sc(stripe):
    b_s = STRIPES[stripe]
    chunk = b_s // NW

    @functools.partial(
        pl.kernel,
        mesh=plsc.VectorSubcoreMesh(core_axis_name="c", subcore_axis_name="s"),
        scratch_types=[
            pltpu.VMEM((chunk,), jnp.int32),
            pltpu.VMEM((chunk, D), jnp.float32),
            pltpu.VMEM((chunk, D), jnp.float32),
            pltpu.SemaphoreType.DMA,
            pltpu.SemaphoreType.DMA,
        ],
    )
    def _gather_add_sc(flat_hbm, idx_hbm, val_hbm, out_hbm, idx_v, rows_v,
                       flat_v, gsem, fsem):
        wid = lax.axis_index("s") * NC + lax.axis_index("c")
        base = wid * chunk
        fcopy = pltpu.async_copy(
            flat_hbm.at[pl.ds(OFFSETS[stripe] + base, chunk)], flat_v, fsem)
        pltpu.sync_copy(idx_hbm.at[pl.ds(base, chunk)], idx_v)
        gather = pltpu.async_copy(val_hbm.at[idx_v], rows_v, gsem)
        fcopy.wait()
        gather.wait()

        @plsc.parallel_loop(0, chunk, 1, unroll=4)
        def _add(r):
            for j in range(D // 16):
                sl = pl.ds(j * 16, 16)
                rows_v[r, sl] = rows_v[r, sl] + flat_v[r, sl]

        pltpu.sync_copy(
            rows_v, out_hbm.at[pl.ds(OFFSETS[stripe] + base, chunk)])

    return _gather_add_sc


def kernel(inputs, key_weights, value_weights):
    size = inputs.shape
    flat = inputs.reshape(-1, D)
    out_ref = jax.new_ref(lax.empty((B, D), jnp.float32))
    idx0, knorm = _argmin_tc_first(flat, key_weights)
    _make_gather_add_sc(0)(flat, idx0, value_weights, out_ref)
    for r in range(1, len(STRIPES)):
        idx = _argmin_tc_rest(flat, key_weights, knorm, r)
        _make_gather_add_sc(r)(flat, idx, value_weights, out_ref)
    return out_ref[...].reshape(size)
